# baseline (speedup 1.0000x reference)
import functools

import jax
import jax.numpy as jnp
from jax import lax
from jax.experimental import pallas as pl
from jax.experimental.pallas import tpu as pltpu
from jax.experimental.pallas import tpu_sc as plsc

BATCH = 16384
EMBED_K = 16
NROWS = 1000000
_info = plsc.get_sparse_core_info()
NC, NS, L = _info.num_cores, _info.num_subcores, _info.num_lanes
NW = NC * NS
BPW = BATCH // NW
IDX_CHUNK = 128
N_CHUNKS = BPW // IDX_CHUNK

TCOLS = (NROWS + 127) // 128
DPITCH = TCOLS * 128
FLAT = EMBED_K * DPITCH
CHUNK_TC = 16


N_FULL = TCOLS // CHUNK_TC
REM_TC = TCOLS - N_FULL * CHUNK_TC


def _detile_body(wt, ht, wflat, hflat, band0, band1,
                 isem0, isem1, osem0, osem1, sem):
    wid = lax.axis_index("s") * NC + lax.axis_index("c")
    ntrips = N_FULL // NW + jnp.where(wid < N_FULL % NW, 1, 0)

    def table(src, dst):
        def lanes_of(k):
            return (wid + k * NW) * (CHUNK_TC * 128)

        def start_read(k, band, isem):
            pltpu.async_copy(
                src.at[:, pl.ds(pl.multiple_of(lanes_of(k), 128),
                                CHUNK_TC * 128)], band, isem)

        def write_out(k, band, osem):
            lo = lanes_of(k)
            for d in range(EMBED_K):
                pltpu.async_copy(
                    band.at[d],
                    dst.at[pl.ds(pl.multiple_of(d * DPITCH, 8) + lo,
                                 CHUNK_TC * 128)], osem)

        def drain(ref, sem_):
            pltpu.make_async_copy(src.at[:, pl.ds(0, CHUNK_TC * 128)],
                                  ref, sem_).wait()

        def nop(_):
            return 0

        start_read(0, band0, isem0)

        def body(k, carry):
            def even(_):
                def wrb(__):
                    drain(band1, osem1)
                    return 0

                def rd(__):
                    start_read(k + 1, band1, isem1)
                    return 0

                drain(band0, isem0)
                lax.cond(k >= 1, wrb, nop, 0)
                lax.cond(k + 1 < ntrips, rd, nop, 0)
                write_out(k, band0, osem0)
                return 0

            def odd(_):
                def wrb(__):
                    drain(band0, osem0)
                    return 0

                def rd(__):
                    start_read(k + 1, band0, isem0)
                    return 0

                drain(band1, isem1)
                lax.cond(k >= 1, wrb, nop, 0)
                lax.cond(k + 1 < ntrips, rd, nop, 0)
                write_out(k, band1, osem1)
                return 0

            lax.cond(k % 2 == 0, even, odd, 0)
            return carry

        lax.fori_loop(0, ntrips, body, 0)

        def dr0(_):
            drain(band0, osem0)
            return 0

        def dr1(_):
            drain(band1, osem1)
            return 0

        lax.cond((ntrips - 1) % 2 == 0, dr0, dr1, 0)

        @pl.when(wid == 0)
        def _():
            lo = N_FULL * CHUNK_TC * 128
            pltpu.async_copy(
                src.at[:, pl.ds(pl.multiple_of(lo, 128), REM_TC * 128)],
                band0.at[:, pl.ds(0, REM_TC * 128)], sem).wait()
            cps = [pltpu.async_copy(
                band0.at[d, pl.ds(0, REM_TC * 128)],
                dst.at[pl.ds(pl.multiple_of(d * DPITCH, 8) + lo,
                             REM_TC * 128)], sem)
                for d in range(EMBED_K)]
            for cp in cps:
                cp.wait()

    table(wt, wflat)
    table(ht, hflat)


@functools.partial(
    pl.kernel,
    out_type=(
        jax.ShapeDtypeStruct((FLAT,), jnp.float32),
        jax.ShapeDtypeStruct((FLAT,), jnp.float32),
    ),
    mesh=plsc.VectorSubcoreMesh(core_axis_name="c", subcore_axis_name="s"),
    compiler_params=pltpu.CompilerParams(
        needs_layout_passes=False, use_tc_tiling_on_sc=True),
    scratch_types=[
        pltpu.VMEM((EMBED_K, CHUNK_TC * 128), jnp.float32),
        pltpu.VMEM((EMBED_K, CHUNK_TC * 128), jnp.float32),
        pltpu.SemaphoreType.DMA,
        pltpu.SemaphoreType.DMA,
        pltpu.SemaphoreType.DMA,
        pltpu.SemaphoreType.DMA,
        pltpu.SemaphoreType.DMA,
    ],
)
def _detile(wt, ht, wflat, hflat, band0, band1,
            isem0, isem1, osem0, osem1, sem):
    _detile_body(wt, ht, wflat, hflat, band0, band1,
                 isem0, isem1, osem0, osem1, sem)


def _gather_body(u_hbm, v_hbm, wf, hf, out_hbm,
                 uidx, vidx, fidx, ut, vt, out_v, sem):
    wid = lax.axis_index("s") * NC + lax.axis_index("c")
    base = wid * BPW

    pltpu.sync_copy(u_hbm.at[pl.ds(wid * N_CHUNKS, N_CHUNKS), :], uidx)
    pltpu.sync_copy(v_hbm.at[pl.ds(wid * N_CHUNKS, N_CHUNKS), :], vidx)

    for c in range(N_CHUNKS):
        for j in range(IDX_CHUNK // L):
            sl = pl.ds(j * L, L)
            iu = uidx[c, sl]
            iv = vidx[c, sl]
            for d in range(EMBED_K):
                fidx[d * N_CHUNKS + c, sl] = iu + d * DPITCH
                fidx[(EMBED_K + d) * N_CHUNKS + c, sl] = iv + d * DPITCH

    copies = []
    for d in range(EMBED_K):
        for c in range(N_CHUNKS):
            copies.append(pltpu.async_copy(
                wf.at[fidx.at[d * N_CHUNKS + c]],
                ut.at[d, pl.ds(c * IDX_CHUNK, IDX_CHUNK)], sem))
            copies.append(pltpu.async_copy(
                hf.at[fidx.at[(EMBED_K + d) * N_CHUNKS + c]],
                vt.at[d, pl.ds(c * IDX_CHUNK, IDX_CHUNK)], sem))
    for cp in copies:
        cp.wait()

    def block(j, carry):
        sl = pl.ds(j * L, L)
        acc = ut[0, sl] * vt[0, sl]
        for d in range(1, EMBED_K):
            acc = acc + ut[d, sl] * vt[d, sl]
        out_v[sl] = 1.0 / (1.0 + jnp.exp(-acc))
        return carry

    lax.fori_loop(0, BPW // L, block, 0)
    pltpu.sync_copy(out_v, out_hbm.at[pl.ds(base, BPW)])


@functools.partial(
    pl.kernel,
    out_type=jax.ShapeDtypeStruct((BATCH,), jnp.float32),
    mesh=plsc.VectorSubcoreMesh(core_axis_name="c", subcore_axis_name="s"),
    compiler_params=pltpu.CompilerParams(
        needs_layout_passes=False, use_tc_tiling_on_sc=False),
    scratch_types=[
        pltpu.VMEM((N_CHUNKS, IDX_CHUNK), jnp.int32),
        pltpu.VMEM((N_CHUNKS, IDX_CHUNK), jnp.int32),
        pltpu.VMEM((2 * EMBED_K * N_CHUNKS, IDX_CHUNK), jnp.int32),
        pltpu.VMEM((EMBED_K, BPW), jnp.float32),
        pltpu.VMEM((EMBED_K, BPW), jnp.float32),
        pltpu.VMEM((BPW,), jnp.float32),
        pltpu.SemaphoreType.DMA,
    ],
)
def _gather_dot(u_hbm, v_hbm, wf, hf, out_hbm,
                uidx, vidx, fidx, ut, vt, out_v, sem):
    _gather_body(u_hbm, v_hbm, wf, hf, out_hbm,
                 uidx, vidx, fidx, ut, vt, out_v, sem)


def kernel(x, W, H):
    u2d = x[:, 0].astype(jnp.int32).reshape(NW * N_CHUNKS, IDX_CHUNK)
    v2d = x[:, 1].astype(jnp.int32).reshape(NW * N_CHUNKS, IDX_CHUNK)
    wflat, hflat = _detile(W.T, H.T)
    return _gather_dot(u2d, v2d, wflat, hflat)


# speedup vs baseline: 1.0086x; 1.0086x over previous
import functools

import jax
import jax.numpy as jnp
from jax import lax
from jax.experimental import pallas as pl
from jax.experimental.pallas import tpu as pltpu
from jax.experimental.pallas import tpu_sc as plsc

BATCH = 16384
EMBED_K = 16
NROWS = 1000000
_info = plsc.get_sparse_core_info()
NC, NS, L = _info.num_cores, _info.num_subcores, _info.num_lanes
NW = NC * NS
BPW = BATCH // NW
IDX_CHUNK = 128
N_CHUNKS = BPW // IDX_CHUNK

TCOLS = (NROWS + 127) // 128
DPITCH = TCOLS * 128
FLAT = EMBED_K * DPITCH
CHUNK_TC = 16


N_FULL = TCOLS // CHUNK_TC
REM_TC = TCOLS - N_FULL * CHUNK_TC


def _detile_body(wt, ht, wflat, hflat, band0, band1,
                 isem0, isem1, osem0, osem1, sem):
    wid = lax.axis_index("s") * NC + lax.axis_index("c")
    ntrips = N_FULL // NW + jnp.where(wid < N_FULL % NW, 1, 0)

    def table(src, dst):
        def lanes_of(k):
            return (wid + k * NW) * (CHUNK_TC * 128)

        def start_read(k, band, isem):
            pltpu.async_copy(
                src.at[:, pl.ds(pl.multiple_of(lanes_of(k), 128),
                                CHUNK_TC * 128)], band, isem)

        def write_out(k, band, osem):
            lo = lanes_of(k)
            for d in range(EMBED_K):
                pltpu.async_copy(
                    band.at[d],
                    dst.at[pl.ds(pl.multiple_of(d * DPITCH, 8) + lo,
                                 CHUNK_TC * 128)], osem)

        def drain(ref, sem_):
            pltpu.make_async_copy(src.at[:, pl.ds(0, CHUNK_TC * 128)],
                                  ref, sem_).wait()

        def nop(_):
            return 0

        start_read(0, band0, isem0)

        def body(k, carry):
            def even(_):
                def wrb(__):
                    drain(band1, osem1)
                    return 0

                def rd(__):
                    start_read(k + 1, band1, isem1)
                    return 0

                drain(band0, isem0)
                lax.cond(k >= 1, wrb, nop, 0)
                lax.cond(k + 1 < ntrips, rd, nop, 0)
                write_out(k, band0, osem0)
                return 0

            def odd(_):
                def wrb(__):
                    drain(band0, osem0)
                    return 0

                def rd(__):
                    start_read(k + 1, band0, isem0)
                    return 0

                drain(band1, isem1)
                lax.cond(k >= 1, wrb, nop, 0)
                lax.cond(k + 1 < ntrips, rd, nop, 0)
                write_out(k, band1, osem1)
                return 0

            lax.cond(k % 2 == 0, even, odd, 0)
            return carry

        lax.fori_loop(0, ntrips, body, 0)

        def dr0(_):
            drain(band0, osem0)
            return 0

        def dr1(_):
            drain(band1, osem1)
            return 0

        lax.cond((ntrips - 1) % 2 == 0, dr0, dr1, 0)

        @pl.when(wid == 0)
        def _():
            lo = N_FULL * CHUNK_TC * 128
            pltpu.async_copy(
                src.at[:, pl.ds(pl.multiple_of(lo, 128), REM_TC * 128)],
                band0.at[:, pl.ds(0, REM_TC * 128)], sem).wait()
            cps = [pltpu.async_copy(
                band0.at[d, pl.ds(0, REM_TC * 128)],
                dst.at[pl.ds(pl.multiple_of(d * DPITCH, 8) + lo,
                             REM_TC * 128)], sem)
                for d in range(EMBED_K)]
            for cp in cps:
                cp.wait()

    table(wt, wflat)
    table(ht, hflat)


@functools.partial(
    pl.kernel,
    out_type=(
        jax.ShapeDtypeStruct((FLAT,), jnp.float32),
        jax.ShapeDtypeStruct((FLAT,), jnp.float32),
    ),
    mesh=plsc.VectorSubcoreMesh(core_axis_name="c", subcore_axis_name="s"),
    compiler_params=pltpu.CompilerParams(
        needs_layout_passes=False, use_tc_tiling_on_sc=True),
    scratch_types=[
        pltpu.VMEM((EMBED_K, CHUNK_TC * 128), jnp.float32),
        pltpu.VMEM((EMBED_K, CHUNK_TC * 128), jnp.float32),
        pltpu.SemaphoreType.DMA,
        pltpu.SemaphoreType.DMA,
        pltpu.SemaphoreType.DMA,
        pltpu.SemaphoreType.DMA,
        pltpu.SemaphoreType.DMA,
    ],
)
def _detile(wt, ht, wflat, hflat, band0, band1,
            isem0, isem1, osem0, osem1, sem):
    _detile_body(wt, ht, wflat, hflat, band0, band1,
                 isem0, isem1, osem0, osem1, sem)


def _gather_body(u_hbm, v_hbm, wf, hf, out_hbm,
                 uidx, vidx, fidx, ut, vt, out_v, sem):
    wid = lax.axis_index("s") * NC + lax.axis_index("c")
    base = wid * BPW

    pltpu.sync_copy(u_hbm.at[pl.ds(wid * N_CHUNKS, N_CHUNKS), :], uidx)
    pltpu.sync_copy(v_hbm.at[pl.ds(wid * N_CHUNKS, N_CHUNKS), :], vidx)

    for c in range(N_CHUNKS):
        for j in range(IDX_CHUNK // L):
            sl = pl.ds(j * L, L)
            iu = uidx[c, sl]
            iv = vidx[c, sl]
            for d in range(EMBED_K):
                fidx[d * N_CHUNKS + c, sl] = iu + d * DPITCH
                fidx[(EMBED_K + d) * N_CHUNKS + c, sl] = iv + d * DPITCH

    for d in range(EMBED_K):
        for c in range(N_CHUNKS):
            pltpu.async_copy(
                wf.at[fidx.at[d * N_CHUNKS + c]],
                ut.at[d, pl.ds(c * IDX_CHUNK, IDX_CHUNK)], sem)
            pltpu.async_copy(
                hf.at[fidx.at[(EMBED_K + d) * N_CHUNKS + c]],
                vt.at[d, pl.ds(c * IDX_CHUNK, IDX_CHUNK)], sem)
    pltpu.make_async_copy(wf.at[pl.ds(0, BPW * EMBED_K)], ut, sem).wait()
    pltpu.make_async_copy(hf.at[pl.ds(0, BPW * EMBED_K)], vt, sem).wait()

    def block(j, carry):
        sl = pl.ds(j * L, L)
        acc = ut[0, sl] * vt[0, sl]
        for d in range(1, EMBED_K):
            acc = acc + ut[d, sl] * vt[d, sl]
        out_v[sl] = 1.0 / (1.0 + jnp.exp(-acc))
        return carry

    lax.fori_loop(0, BPW // L, block, 0)
    pltpu.sync_copy(out_v, out_hbm.at[pl.ds(base, BPW)])


@functools.partial(
    pl.kernel,
    out_type=jax.ShapeDtypeStruct((BATCH,), jnp.float32),
    mesh=plsc.VectorSubcoreMesh(core_axis_name="c", subcore_axis_name="s"),
    compiler_params=pltpu.CompilerParams(
        needs_layout_passes=False, use_tc_tiling_on_sc=False),
    scratch_types=[
        pltpu.VMEM((N_CHUNKS, IDX_CHUNK), jnp.int32),
        pltpu.VMEM((N_CHUNKS, IDX_CHUNK), jnp.int32),
        pltpu.VMEM((2 * EMBED_K * N_CHUNKS, IDX_CHUNK), jnp.int32),
        pltpu.VMEM((EMBED_K, BPW), jnp.float32),
        pltpu.VMEM((EMBED_K, BPW), jnp.float32),
        pltpu.VMEM((BPW,), jnp.float32),
        pltpu.SemaphoreType.DMA,
    ],
)
def _gather_dot(u_hbm, v_hbm, wf, hf, out_hbm,
                uidx, vidx, fidx, ut, vt, out_v, sem):
    _gather_body(u_hbm, v_hbm, wf, hf, out_hbm,
                 uidx, vidx, fidx, ut, vt, out_v, sem)


def kernel(x, W, H):
    u2d = x[:, 0].astype(jnp.int32).reshape(NW * N_CHUNKS, IDX_CHUNK)
    v2d = x[:, 1].astype(jnp.int32).reshape(NW * N_CHUNKS, IDX_CHUNK)
    wflat, hflat = _detile(W.T, H.T)
    return _gather_dot(u2d, v2d, wflat, hflat)


# speedup vs baseline: 1.0241x; 1.0153x over previous
import functools

import jax
import jax.numpy as jnp
from jax import lax
from jax.experimental import pallas as pl
from jax.experimental.pallas import tpu as pltpu
from jax.experimental.pallas import tpu_sc as plsc

BATCH = 16384
EMBED_K = 16
NROWS = 1000000
_info = plsc.get_sparse_core_info()
NC, NS, L = _info.num_cores, _info.num_subcores, _info.num_lanes
NW = NC * NS
BPW = BATCH // NW
IDX_CHUNK = 128
N_CHUNKS = BPW // IDX_CHUNK

TCOLS = (NROWS + 127) // 128
DPITCH = TCOLS * 128
FLAT = EMBED_K * DPITCH
CHUNK_TC = 24


N_FULL = TCOLS // CHUNK_TC
REM_TC = TCOLS - N_FULL * CHUNK_TC


def _detile_body(wt, ht, wflat, hflat, band0, band1,
                 isem0, isem1, osem0, osem1, sem):
    wid = lax.axis_index("s") * NC + lax.axis_index("c")
    ntrips = N_FULL // NW + jnp.where(wid < N_FULL % NW, 1, 0)

    def table(src, dst):
        def lanes_of(k):
            return (wid + k * NW) * (CHUNK_TC * 128)

        def start_read(k, band, isem):
            pltpu.async_copy(
                src.at[:, pl.ds(pl.multiple_of(lanes_of(k), 128),
                                CHUNK_TC * 128)], band, isem)

        def write_out(k, band, osem):
            lo = lanes_of(k)
            for d in range(EMBED_K):
                pltpu.async_copy(
                    band.at[d],
                    dst.at[pl.ds(pl.multiple_of(d * DPITCH, 8) + lo,
                                 CHUNK_TC * 128)], osem)

        def drain(ref, sem_):
            pltpu.make_async_copy(src.at[:, pl.ds(0, CHUNK_TC * 128)],
                                  ref, sem_).wait()

        def nop(_):
            return 0

        start_read(0, band0, isem0)

        def body(k, carry):
            def even(_):
                def wrb(__):
                    drain(band1, osem1)
                    return 0

                def rd(__):
                    start_read(k + 1, band1, isem1)
                    return 0

                drain(band0, isem0)
                lax.cond(k >= 1, wrb, nop, 0)
                lax.cond(k + 1 < ntrips, rd, nop, 0)
                write_out(k, band0, osem0)
                return 0

            def odd(_):
                def wrb(__):
                    drain(band0, osem0)
                    return 0

                def rd(__):
                    start_read(k + 1, band0, isem0)
                    return 0

                drain(band1, isem1)
                lax.cond(k >= 1, wrb, nop, 0)
                lax.cond(k + 1 < ntrips, rd, nop, 0)
                write_out(k, band1, osem1)
                return 0

            lax.cond(k % 2 == 0, even, odd, 0)
            return carry

        lax.fori_loop(0, ntrips, body, 0)

        def dr0(_):
            drain(band0, osem0)
            return 0

        def dr1(_):
            drain(band1, osem1)
            return 0

        lax.cond((ntrips - 1) % 2 == 0, dr0, dr1, 0)

        @pl.when(wid == 0)
        def _():
            lo = N_FULL * CHUNK_TC * 128
            pltpu.async_copy(
                src.at[:, pl.ds(pl.multiple_of(lo, 128), REM_TC * 128)],
                band0.at[:, pl.ds(0, REM_TC * 128)], sem).wait()
            cps = [pltpu.async_copy(
                band0.at[d, pl.ds(0, REM_TC * 128)],
                dst.at[pl.ds(pl.multiple_of(d * DPITCH, 8) + lo,
                             REM_TC * 128)], sem)
                for d in range(EMBED_K)]
            for cp in cps:
                cp.wait()

    table(wt, wflat)
    table(ht, hflat)


@functools.partial(
    pl.kernel,
    out_type=(
        jax.ShapeDtypeStruct((FLAT,), jnp.float32),
        jax.ShapeDtypeStruct((FLAT,), jnp.float32),
    ),
    mesh=plsc.VectorSubcoreMesh(core_axis_name="c", subcore_axis_name="s"),
    compiler_params=pltpu.CompilerParams(
        needs_layout_passes=False, use_tc_tiling_on_sc=True),
    scratch_types=[
        pltpu.VMEM((EMBED_K, CHUNK_TC * 128), jnp.float32),
        pltpu.VMEM((EMBED_K, CHUNK_TC * 128), jnp.float32),
        pltpu.SemaphoreType.DMA,
        pltpu.SemaphoreType.DMA,
        pltpu.SemaphoreType.DMA,
        pltpu.SemaphoreType.DMA,
        pltpu.SemaphoreType.DMA,
    ],
)
def _detile(wt, ht, wflat, hflat, band0, band1,
            isem0, isem1, osem0, osem1, sem):
    _detile_body(wt, ht, wflat, hflat, band0, band1,
                 isem0, isem1, osem0, osem1, sem)


def _gather_body(u_hbm, v_hbm, wf, hf, out_hbm,
                 uidx, vidx, fidx, ut, vt, out_v, sem):
    wid = lax.axis_index("s") * NC + lax.axis_index("c")
    base = wid * BPW

    pltpu.sync_copy(u_hbm.at[pl.ds(wid * N_CHUNKS, N_CHUNKS), :], uidx)
    pltpu.sync_copy(v_hbm.at[pl.ds(wid * N_CHUNKS, N_CHUNKS), :], vidx)

    for c in range(N_CHUNKS):
        for j in range(IDX_CHUNK // L):
            sl = pl.ds(j * L, L)
            iu = uidx[c, sl]
            iv = vidx[c, sl]
            for d in range(EMBED_K):
                fidx[d * N_CHUNKS + c, sl] = iu + d * DPITCH
                fidx[(EMBED_K + d) * N_CHUNKS + c, sl] = iv + d * DPITCH

    for d in range(EMBED_K):
        for c in range(N_CHUNKS):
            pltpu.async_copy(
                wf.at[fidx.at[d * N_CHUNKS + c]],
                ut.at[d, pl.ds(c * IDX_CHUNK, IDX_CHUNK)], sem)
            pltpu.async_copy(
                hf.at[fidx.at[(EMBED_K + d) * N_CHUNKS + c]],
                vt.at[d, pl.ds(c * IDX_CHUNK, IDX_CHUNK)], sem)
    pltpu.make_async_copy(wf.at[pl.ds(0, BPW * EMBED_K)], ut, sem).wait()
    pltpu.make_async_copy(hf.at[pl.ds(0, BPW * EMBED_K)], vt, sem).wait()

    def block(j, carry):
        sl = pl.ds(j * L, L)
        acc = ut[0, sl] * vt[0, sl]
        for d in range(1, EMBED_K):
            acc = acc + ut[d, sl] * vt[d, sl]
        out_v[sl] = 1.0 / (1.0 + jnp.exp(-acc))
        return carry

    lax.fori_loop(0, BPW // L, block, 0)
    pltpu.sync_copy(out_v, out_hbm.at[pl.ds(base, BPW)])


@functools.partial(
    pl.kernel,
    out_type=jax.ShapeDtypeStruct((BATCH,), jnp.float32),
    mesh=plsc.VectorSubcoreMesh(core_axis_name="c", subcore_axis_name="s"),
    compiler_params=pltpu.CompilerParams(
        needs_layout_passes=False, use_tc_tiling_on_sc=False),
    scratch_types=[
        pltpu.VMEM((N_CHUNKS, IDX_CHUNK), jnp.int32),
        pltpu.VMEM((N_CHUNKS, IDX_CHUNK), jnp.int32),
        pltpu.VMEM((2 * EMBED_K * N_CHUNKS, IDX_CHUNK), jnp.int32),
        pltpu.VMEM((EMBED_K, BPW), jnp.float32),
        pltpu.VMEM((EMBED_K, BPW), jnp.float32),
        pltpu.VMEM((BPW,), jnp.float32),
        pltpu.SemaphoreType.DMA,
    ],
)
def _gather_dot(u_hbm, v_hbm, wf, hf, out_hbm,
                uidx, vidx, fidx, ut, vt, out_v, sem):
    _gather_body(u_hbm, v_hbm, wf, hf, out_hbm,
                 uidx, vidx, fidx, ut, vt, out_v, sem)


def kernel(x, W, H):
    u2d = x[:, 0].astype(jnp.int32).reshape(NW * N_CHUNKS, IDX_CHUNK)
    v2d = x[:, 1].astype(jnp.int32).reshape(NW * N_CHUNKS, IDX_CHUNK)
    wflat, hflat = _detile(W.T, H.T)
    return _gather_dot(u2d, v2d, wflat, hflat)


# speedup vs baseline: 1.0855x; 1.0600x over previous
import functools

import jax
import jax.numpy as jnp
from jax import lax
from jax.experimental import pallas as pl
from jax.experimental.pallas import tpu as pltpu
from jax.experimental.pallas import tpu_sc as plsc

BATCH = 16384
EMBED_K = 16
NROWS = 1000000
_info = plsc.get_sparse_core_info()
NC, NS, L = _info.num_cores, _info.num_subcores, _info.num_lanes
NW = NC * NS
BPW = BATCH // NW
IDX_CHUNK = 128
N_CHUNKS = BPW // IDX_CHUNK

TCOLS = (NROWS + 127) // 128
DPITCH = TCOLS * 128
FLAT = EMBED_K * DPITCH
CHUNK_TC = 31


N_FULL = TCOLS // CHUNK_TC
REM_TC = TCOLS - N_FULL * CHUNK_TC


def _detile_body(wt, ht, wflat, hflat, band0, band1,
                 isem0, isem1, osem0, osem1, sem):
    wid = lax.axis_index("s") * NC + lax.axis_index("c")
    ntrips = N_FULL // NW + jnp.where(wid < N_FULL % NW, 1, 0)

    def table(src, dst):
        def lanes_of(k):
            return (wid + k * NW) * (CHUNK_TC * 128)

        def start_read(k, band, isem):
            pltpu.async_copy(
                src.at[:, pl.ds(pl.multiple_of(lanes_of(k), 128),
                                CHUNK_TC * 128)], band, isem)

        def write_out(k, band, osem):
            lo = lanes_of(k)
            for d in range(EMBED_K):
                pltpu.async_copy(
                    band.at[d],
                    dst.at[pl.ds(pl.multiple_of(d * DPITCH, 8) + lo,
                                 CHUNK_TC * 128)], osem)

        def drain(ref, sem_):
            pltpu.make_async_copy(src.at[:, pl.ds(0, CHUNK_TC * 128)],
                                  ref, sem_).wait()

        def nop(_):
            return 0

        start_read(0, band0, isem0)

        def body(k, carry):
            def even(_):
                def wrb(__):
                    drain(band1, osem1)
                    return 0

                def rd(__):
                    start_read(k + 1, band1, isem1)
                    return 0

                drain(band0, isem0)
                lax.cond(k >= 1, wrb, nop, 0)
                lax.cond(k + 1 < ntrips, rd, nop, 0)
                write_out(k, band0, osem0)
                return 0

            def odd(_):
                def wrb(__):
                    drain(band0, osem0)
                    return 0

                def rd(__):
                    start_read(k + 1, band0, isem0)
                    return 0

                drain(band1, isem1)
                lax.cond(k >= 1, wrb, nop, 0)
                lax.cond(k + 1 < ntrips, rd, nop, 0)
                write_out(k, band1, osem1)
                return 0

            lax.cond(k % 2 == 0, even, odd, 0)
            return carry

        lax.fori_loop(0, ntrips, body, 0)

        def dr0(_):
            drain(band0, osem0)
            return 0

        def dr1(_):
            drain(band1, osem1)
            return 0

        lax.cond((ntrips - 1) % 2 == 0, dr0, dr1, 0)

        @pl.when(wid == 0)
        def _():
            lo = N_FULL * CHUNK_TC * 128
            pltpu.async_copy(
                src.at[:, pl.ds(pl.multiple_of(lo, 128), REM_TC * 128)],
                band0.at[:, pl.ds(0, REM_TC * 128)], sem).wait()
            cps = [pltpu.async_copy(
                band0.at[d, pl.ds(0, REM_TC * 128)],
                dst.at[pl.ds(pl.multiple_of(d * DPITCH, 8) + lo,
                             REM_TC * 128)], sem)
                for d in range(EMBED_K)]
            for cp in cps:
                cp.wait()

    table(wt, wflat)
    table(ht, hflat)


@functools.partial(
    pl.kernel,
    out_type=(
        jax.ShapeDtypeStruct((FLAT,), jnp.float32),
        jax.ShapeDtypeStruct((FLAT,), jnp.float32),
    ),
    mesh=plsc.VectorSubcoreMesh(core_axis_name="c", subcore_axis_name="s"),
    compiler_params=pltpu.CompilerParams(
        needs_layout_passes=False, use_tc_tiling_on_sc=True),
    scratch_types=[
        pltpu.VMEM((EMBED_K, CHUNK_TC * 128), jnp.float32),
        pltpu.VMEM((EMBED_K, CHUNK_TC * 128), jnp.float32),
        pltpu.SemaphoreType.DMA,
        pltpu.SemaphoreType.DMA,
        pltpu.SemaphoreType.DMA,
        pltpu.SemaphoreType.DMA,
        pltpu.SemaphoreType.DMA,
    ],
)
def _detile(wt, ht, wflat, hflat, band0, band1,
            isem0, isem1, osem0, osem1, sem):
    _detile_body(wt, ht, wflat, hflat, band0, band1,
                 isem0, isem1, osem0, osem1, sem)


def _gather_body(u_hbm, v_hbm, wf, hf, out_hbm,
                 uidx, vidx, fidx, ut, vt, out_v, sem):
    wid = lax.axis_index("s") * NC + lax.axis_index("c")
    base = wid * BPW

    pltpu.sync_copy(u_hbm.at[pl.ds(wid * N_CHUNKS, N_CHUNKS), :], uidx)
    pltpu.sync_copy(v_hbm.at[pl.ds(wid * N_CHUNKS, N_CHUNKS), :], vidx)

    for c in range(N_CHUNKS):
        for j in range(IDX_CHUNK // L):
            sl = pl.ds(j * L, L)
            iu = uidx[c, sl]
            iv = vidx[c, sl]
            for d in range(EMBED_K):
                fidx[d * N_CHUNKS + c, sl] = iu + d * DPITCH
                fidx[(EMBED_K + d) * N_CHUNKS + c, sl] = iv + d * DPITCH

    for d in range(EMBED_K):
        for c in range(N_CHUNKS):
            pltpu.async_copy(
                wf.at[fidx.at[d * N_CHUNKS + c]],
                ut.at[d, pl.ds(c * IDX_CHUNK, IDX_CHUNK)], sem)
            pltpu.async_copy(
                hf.at[fidx.at[(EMBED_K + d) * N_CHUNKS + c]],
                vt.at[d, pl.ds(c * IDX_CHUNK, IDX_CHUNK)], sem)
    pltpu.make_async_copy(wf.at[pl.ds(0, BPW * EMBED_K)], ut, sem).wait()
    pltpu.make_async_copy(hf.at[pl.ds(0, BPW * EMBED_K)], vt, sem).wait()

    def block(j, carry):
        sl = pl.ds(j * L, L)
        acc = ut[0, sl] * vt[0, sl]
        for d in range(1, EMBED_K):
            acc = acc + ut[d, sl] * vt[d, sl]
        out_v[sl] = 1.0 / (1.0 + jnp.exp(-acc))
        return carry

    lax.fori_loop(0, BPW // L, block, 0)
    pltpu.sync_copy(out_v, out_hbm.at[pl.ds(base, BPW)])


@functools.partial(
    pl.kernel,
    out_type=jax.ShapeDtypeStruct((BATCH,), jnp.float32),
    mesh=plsc.VectorSubcoreMesh(core_axis_name="c", subcore_axis_name="s"),
    compiler_params=pltpu.CompilerParams(
        needs_layout_passes=False, use_tc_tiling_on_sc=False),
    scratch_types=[
        pltpu.VMEM((N_CHUNKS, IDX_CHUNK), jnp.int32),
        pltpu.VMEM((N_CHUNKS, IDX_CHUNK), jnp.int32),
        pltpu.VMEM((2 * EMBED_K * N_CHUNKS, IDX_CHUNK), jnp.int32),
        pltpu.VMEM((EMBED_K, BPW), jnp.float32),
        pltpu.VMEM((EMBED_K, BPW), jnp.float32),
        pltpu.VMEM((BPW,), jnp.float32),
        pltpu.SemaphoreType.DMA,
    ],
)
def _gather_dot(u_hbm, v_hbm, wf, hf, out_hbm,
                uidx, vidx, fidx, ut, vt, out_v, sem):
    _gather_body(u_hbm, v_hbm, wf, hf, out_hbm,
                 uidx, vidx, fidx, ut, vt, out_v, sem)


def kernel(x, W, H):
    u2d = x[:, 0].astype(jnp.int32).reshape(NW * N_CHUNKS, IDX_CHUNK)
    v2d = x[:, 1].astype(jnp.int32).reshape(NW * N_CHUNKS, IDX_CHUNK)
    wflat, hflat = _detile(W.T, H.T)
    return _gather_dot(u2d, v2d, wflat, hflat)
